# baseline (speedup 1.0000x reference)
import functools

import jax
import jax.numpy as jnp
from jax import lax
from jax.experimental import pallas as pl
from jax.experimental.pallas import tpu as pltpu
from jax.experimental.pallas import tpu_sc as plsc

B = 4096
L = 200
V = 1000000
D = 64
H = 256
O = 64

NC = 2
NS = 16
NW = NC * NS
LPAD = 208
HALF = LPAD // 2
ROWS_PER_W = B // NW
NBUF = 2

_DIMNUMS = lax.GatherDimensionNumbers(
    offset_dims=(), collapsed_slice_dims=(0,), start_index_map=(0,))


def _lane_splat(x16, i):
    idx = jnp.full((16, 1), i, jnp.int32)
    return lax.gather(x16, idx, _DIMNUMS, (1,),
                      mode=lax.GatherScatterMode.PROMISE_IN_BOUNDS)


def _pool_body(inp_hbm, emb_hbm, out_hbm, idx_v, bufs, stage, sems):
    wid = lax.axis_index("s") * NC + lax.axis_index("c")
    rbase = wid * ROWS_PER_W

    pltpu.sync_copy(inp_hbm.at[pl.ds(rbase * LPAD, ROWS_PER_W * LPAD)], idx_v)

    def fire(r, b):
        base = pl.multiple_of(r * LPAD, LPAD)
        for k in range(LPAD // 16):
            qvec = lax.shift_right_logical(idx_v[pl.ds(base + 16 * k, 16)], 1)
            pltpu.async_copy(emb_hbm.at[qvec],
                             bufs.at[b, pl.ds(16 * k, 16)], sems.at[b])

    def wait(b):
        pltpu.make_async_copy(emb_hbm.at[pl.ds(0, LPAD)], bufs.at[b],
                              sems.at[b]).wait()

    for b in range(NBUF):
        fire(b, b)

    cvecs = [jnp.arange(16, dtype=jnp.int32) + 16 * j for j in range(4)]
    NG = ROWS_PER_W // NBUF

    def grp_loop(g, carry):
        for b in range(NBUF):
            r = g * NBUF + b
            ibase = pl.multiple_of(r * LPAD, LPAD)
            wait(b)

            acc = (jnp.zeros((16,), jnp.float32),) * 4
            for k in range(LPAD // 16):
                vv = idx_v[pl.ds(ibase + 16 * k, 16)]
                selv = lax.shift_left(jnp.bitwise_and(vv, 1), 6)

                def acc16(i, acc, selv=selv, k=k, b=b):
                    sspl = _lane_splat(selv, i)
                    tspl = jnp.full((16,), i, jnp.int32) + 16 * k
                    return tuple(
                        acc[j] + plsc.load_gather(bufs.at[b],
                                                  [tspl, sspl + cvecs[j]])
                        for j in range(4))

                acc = lax.fori_loop(0, 16, acc16, acc)

            obase = pl.multiple_of(r * D, D)
            for j in range(4):
                stage[pl.ds(obase + 16 * j, 16)] = acc[j]

            @pl.when(g < NG - 1)
            def _():
                fire(r + NBUF, b)
        return carry

    lax.fori_loop(0, NG, grp_loop, 0)

    pltpu.sync_copy(stage, out_hbm.at[pl.ds(rbase * D, ROWS_PER_W * D)])


_pool = functools.partial(
    pl.kernel,
    out_type=jax.ShapeDtypeStruct((B * D,), jnp.float32),
    mesh=plsc.VectorSubcoreMesh(core_axis_name="c", subcore_axis_name="s"),
    scratch_types=[
        pltpu.VMEM((ROWS_PER_W * LPAD,), jnp.int32),
        pltpu.VMEM((NBUF, LPAD, 2 * D), jnp.float32),
        pltpu.VMEM((ROWS_PER_W * D,), jnp.float32),
        pltpu.SemaphoreType.DMA((NBUF,)),
    ],
    compiler_params=pltpu.CompilerParams(needs_layout_passes=False),
)(_pool_body)


MLP_BLK = 512


def _mlp_body(sums_ref, inp_ref, sl_ref, pad_ref, emb0_ref, w1_ref, b1_ref,
              w2_ref, b2_ref, out_ref):
    is_pad = (inp_ref[...] == pad_ref[...]).astype(jnp.float32)
    cnt = jnp.sum(is_pad, axis=1, keepdims=True) + float(LPAD - L)
    avg = (sums_ref[...] - cnt * emb0_ref[...]) / sl_ref[...]
    h = jnp.dot(avg, w1_ref[...], preferred_element_type=jnp.float32)
    h = jnp.maximum(h + b1_ref[...], 0.0)
    out = jnp.dot(h, w2_ref[...], preferred_element_type=jnp.float32)
    out_ref[...] = out + b2_ref[...]


def _mlp(sums, inp, sl, pad, emb0, w1, b1, w2, b2):
    grid = (B // MLP_BLK,)
    return pl.pallas_call(
        _mlp_body,
        grid=grid,
        in_specs=[
            pl.BlockSpec((MLP_BLK, D), lambda i: (i, 0)),
            pl.BlockSpec((MLP_BLK, L), lambda i: (i, 0)),
            pl.BlockSpec((MLP_BLK, 1), lambda i: (i, 0)),
            pl.BlockSpec((1, 1), lambda i: (0, 0)),
            pl.BlockSpec((1, D), lambda i: (0, 0)),
            pl.BlockSpec((D, H), lambda i: (0, 0)),
            pl.BlockSpec((1, H), lambda i: (0, 0)),
            pl.BlockSpec((H, O), lambda i: (0, 0)),
            pl.BlockSpec((1, O), lambda i: (0, 0)),
        ],
        out_specs=pl.BlockSpec((MLP_BLK, O), lambda i: (i, 0)),
        out_shape=jax.ShapeDtypeStruct((B, O), jnp.float32),
    )(sums, inp, sl, pad, emb0, w1, b1, w2, b2)


def kernel(input, seq_lengths, pad_id, emb, W1, b1, W2, b2):
    pad_arr = jnp.asarray(pad_id, jnp.int32)
    inp_pad = jnp.pad(input, ((0, 0), (0, LPAD - L)),
                      constant_values=pad_arr).reshape(B * LPAD)
    emb_pairs = emb.reshape(V // 2, 2 * D)
    sums = _pool(inp_pad, emb_pairs).reshape(B, D)
    sl = seq_lengths.astype(jnp.float32).reshape(B, 1)
    emb0 = lax.dynamic_slice_in_dim(emb, pad_arr, 1, axis=0)
    return _mlp(sums, input, sl, pad_arr.reshape(1, 1), emb0,
                W1, b1.reshape(1, H), W2, b2.reshape(1, O))


# speedup vs baseline: 1.5243x; 1.5243x over previous
import functools

import jax
import jax.numpy as jnp
from jax import lax
from jax.experimental import pallas as pl
from jax.experimental.pallas import tpu as pltpu
from jax.experimental.pallas import tpu_sc as plsc

B = 4096
L = 200
V = 1000000
D = 64
H = 256
O = 64

NC = 2
NS = 16
NW = NC * NS
LPAD = 208
ROWS_PER_W = B // NW
NBUF = 2


def _pool_body(inp_hbm, emb_hbm, out_hbm, idx_v, bufs, stage, sems):
    wid = lax.axis_index("s") * NC + lax.axis_index("c")
    rbase = wid * ROWS_PER_W

    pltpu.sync_copy(inp_hbm.at[pl.ds(rbase, ROWS_PER_W)], idx_v)

    def fire(r, b):
        for k in range(LPAD // 16):
            vv = idx_v[r, pl.ds(16 * k, 16)]
            for i in range(16):
                pltpu.async_copy(emb_hbm.at[pl.ds(vv[i], 1)],
                                 bufs.at[b, pl.ds(16 * k + i, 1)],
                                 sems.at[b])

    def wait(b):
        pltpu.make_async_copy(emb_hbm.at[pl.ds(0, LPAD)], bufs.at[b],
                              sems.at[b]).wait()

    for b in range(NBUF):
        fire(b, b)

    def acc_row(b):
        def body(t, acc):
            return tuple(acc[j] + bufs[b, t, pl.ds(16 * j, 16)]
                         for j in range(4))
        zero = jnp.zeros((16,), jnp.float32)
        return lax.fori_loop(0, LPAD, body, (zero,) * 4, unroll=8)

    NG = ROWS_PER_W // NBUF

    def grp_loop(g, carry):
        for b in range(NBUF):
            r = g * NBUF + b
            wait(b)
            acc = acc_row(b)
            for j in range(4):
                stage[r, pl.ds(16 * j, 16)] = acc[j]

            @pl.when(g < NG - 1)
            def _():
                fire(r + NBUF, b)
        return carry

    lax.fori_loop(0, NG, grp_loop, 0)

    pltpu.sync_copy(stage, out_hbm.at[pl.ds(rbase, ROWS_PER_W)])


_pool = functools.partial(
    pl.kernel,
    out_type=jax.ShapeDtypeStruct((B, D), jnp.float32),
    mesh=plsc.VectorSubcoreMesh(core_axis_name="c", subcore_axis_name="s"),
    scratch_types=[
        pltpu.VMEM((ROWS_PER_W, LPAD), jnp.int32),
        pltpu.VMEM((NBUF, LPAD, D), jnp.float32),
        pltpu.VMEM((ROWS_PER_W, D), jnp.float32),
        pltpu.SemaphoreType.DMA((NBUF,)),
    ],
    compiler_params=pltpu.CompilerParams(use_tc_tiling_on_sc=False),
)(_pool_body)


MLP_BLK = 512


def _mlp_body(sums_ref, inp_ref, sl_ref, pad_ref, emb0_ref, w1_ref, b1_ref,
              w2_ref, b2_ref, out_ref):
    is_pad = (inp_ref[...] == pad_ref[...]).astype(jnp.float32)
    cnt = jnp.sum(is_pad, axis=1, keepdims=True) + float(LPAD - L)
    avg = (sums_ref[...] - cnt * emb0_ref[...]) / sl_ref[...]
    h = jnp.dot(avg, w1_ref[...], preferred_element_type=jnp.float32)
    h = jnp.maximum(h + b1_ref[...], 0.0)
    out = jnp.dot(h, w2_ref[...], preferred_element_type=jnp.float32)
    out_ref[...] = out + b2_ref[...]


def _mlp(sums, inp, sl, pad, emb0, w1, b1, w2, b2):
    grid = (B // MLP_BLK,)
    return pl.pallas_call(
        _mlp_body,
        grid=grid,
        in_specs=[
            pl.BlockSpec((MLP_BLK, D), lambda i: (i, 0)),
            pl.BlockSpec((MLP_BLK, L), lambda i: (i, 0)),
            pl.BlockSpec((MLP_BLK, 1), lambda i: (i, 0)),
            pl.BlockSpec((1, 1), lambda i: (0, 0)),
            pl.BlockSpec((1, D), lambda i: (0, 0)),
            pl.BlockSpec((D, H), lambda i: (0, 0)),
            pl.BlockSpec((1, H), lambda i: (0, 0)),
            pl.BlockSpec((H, O), lambda i: (0, 0)),
            pl.BlockSpec((1, O), lambda i: (0, 0)),
        ],
        out_specs=pl.BlockSpec((MLP_BLK, O), lambda i: (i, 0)),
        out_shape=jax.ShapeDtypeStruct((B, O), jnp.float32),
    )(sums, inp, sl, pad, emb0, w1, b1, w2, b2)


def kernel(input, seq_lengths, pad_id, emb, W1, b1, W2, b2):
    pad_arr = jnp.asarray(pad_id, jnp.int32)
    inp_pad = jnp.pad(input, ((0, 0), (0, LPAD - L)), constant_values=pad_arr)
    sums = _pool(inp_pad, emb)
    sl = seq_lengths.astype(jnp.float32).reshape(B, 1)
    emb0 = lax.dynamic_slice_in_dim(emb, pad_arr, 1, axis=0)
    return _mlp(sums, input, sl, pad_arr.reshape(1, 1), emb0,
                W1, b1.reshape(1, H), W2, b2.reshape(1, O))


# speedup vs baseline: 1.5260x; 1.0011x over previous
import functools

import jax
import jax.numpy as jnp
from jax import lax
from jax.experimental import pallas as pl
from jax.experimental.pallas import tpu as pltpu
from jax.experimental.pallas import tpu_sc as plsc

B = 4096
L = 200
V = 1000000
D = 64
H = 256
O = 64

NC = 2
NS = 16
NW = NC * NS
LPAD = 208
ROWS_PER_W = B // NW
NBUF = 4


def _pool_body(inp_hbm, emb_hbm, out_hbm, idx_v, bufs, stage, sems):
    wid = lax.axis_index("s") * NC + lax.axis_index("c")
    rbase = wid * ROWS_PER_W

    pltpu.sync_copy(inp_hbm.at[pl.ds(rbase, ROWS_PER_W)], idx_v)

    def fire(r, b):
        pltpu.async_copy(emb_hbm.at[idx_v.at[r]], bufs.at[b], sems.at[b])

    def wait(b):
        pltpu.make_async_copy(emb_hbm.at[idx_v.at[0]], bufs.at[b],
                              sems.at[b]).wait()

    for b in range(NBUF):
        fire(b, b)

    def acc_row(b):
        def body(t, acc):
            return tuple(acc[j] + bufs[b, t, pl.ds(16 * j, 16)]
                         for j in range(4))
        zero = jnp.zeros((16,), jnp.float32)
        return lax.fori_loop(0, LPAD, body, (zero,) * 4, unroll=8)

    NG = ROWS_PER_W // NBUF

    def grp_loop(g, carry):
        for b in range(NBUF):
            r = g * NBUF + b
            wait(b)
            acc = acc_row(b)
            for j in range(4):
                stage[r, pl.ds(16 * j, 16)] = acc[j]

            @pl.when(g < NG - 1)
            def _():
                fire(r + NBUF, b)
        return carry

    lax.fori_loop(0, NG, grp_loop, 0)

    pltpu.sync_copy(stage, out_hbm.at[pl.ds(rbase, ROWS_PER_W)])


_pool = functools.partial(
    pl.kernel,
    out_type=jax.ShapeDtypeStruct((B, D), jnp.float32),
    mesh=plsc.VectorSubcoreMesh(core_axis_name="c", subcore_axis_name="s"),
    scratch_types=[
        pltpu.VMEM((ROWS_PER_W, LPAD), jnp.int32),
        pltpu.VMEM((NBUF, LPAD, D), jnp.float32),
        pltpu.VMEM((ROWS_PER_W, D), jnp.float32),
        pltpu.SemaphoreType.DMA((NBUF,)),
    ],
    compiler_params=pltpu.CompilerParams(use_tc_tiling_on_sc=False),
)(_pool_body)


MLP_BLK = 512


def _mlp_body(sums_ref, inp_ref, sl_ref, pad_ref, emb0_ref, w1_ref, b1_ref,
              w2_ref, b2_ref, out_ref):
    is_pad = (inp_ref[...] == pad_ref[...]).astype(jnp.float32)
    cnt = jnp.sum(is_pad, axis=1, keepdims=True) + float(LPAD - L)
    avg = (sums_ref[...] - cnt * emb0_ref[...]) / sl_ref[...]
    h = jnp.dot(avg, w1_ref[...], preferred_element_type=jnp.float32)
    h = jnp.maximum(h + b1_ref[...], 0.0)
    out = jnp.dot(h, w2_ref[...], preferred_element_type=jnp.float32)
    out_ref[...] = out + b2_ref[...]


def _mlp(sums, inp, sl, pad, emb0, w1, b1, w2, b2):
    grid = (B // MLP_BLK,)
    return pl.pallas_call(
        _mlp_body,
        grid=grid,
        in_specs=[
            pl.BlockSpec((MLP_BLK, D), lambda i: (i, 0)),
            pl.BlockSpec((MLP_BLK, L), lambda i: (i, 0)),
            pl.BlockSpec((MLP_BLK, 1), lambda i: (i, 0)),
            pl.BlockSpec((1, 1), lambda i: (0, 0)),
            pl.BlockSpec((1, D), lambda i: (0, 0)),
            pl.BlockSpec((D, H), lambda i: (0, 0)),
            pl.BlockSpec((1, H), lambda i: (0, 0)),
            pl.BlockSpec((H, O), lambda i: (0, 0)),
            pl.BlockSpec((1, O), lambda i: (0, 0)),
        ],
        out_specs=pl.BlockSpec((MLP_BLK, O), lambda i: (i, 0)),
        out_shape=jax.ShapeDtypeStruct((B, O), jnp.float32),
    )(sums, inp, sl, pad, emb0, w1, b1, w2, b2)


def kernel(input, seq_lengths, pad_id, emb, W1, b1, W2, b2):
    pad_arr = jnp.asarray(pad_id, jnp.int32)
    inp_pad = jnp.pad(input, ((0, 0), (0, LPAD - L)), constant_values=pad_arr)
    sums = _pool(inp_pad, emb)
    sl = seq_lengths.astype(jnp.float32).reshape(B, 1)
    emb0 = lax.dynamic_slice_in_dim(emb, pad_arr, 1, axis=0)
    return _mlp(sums, input, sl, pad_arr.reshape(1, 1), emb0,
                W1, b1.reshape(1, H), W2, b2.reshape(1, O))
